# Initial kernel scaffold; baseline (speedup 1.0000x reference)
import functools

import jax
import jax.numpy as jnp
from jax import lax
from jax.experimental import pallas as pl
from jax.experimental.pallas import tpu as pltpu
import jax.experimental.pallas.tpu_sc as plsc

_N = 10000
_E = 320000
_D = 128
_G = 128
_NLAYERS = 3

_NC = 2
_NS = 16
_NW = _NC * _NS
_CHUNK = 128
_NCHUNK = _E // _CHUNK
_FULL_ITERS = _NCHUNK // _NW
_REM = _NCHUNK - _FULL_ITERS * _NW
_RPT = _N // _NS


def _zero_vmem_2d(ref, rows, cols):
  def body(r, carry):
    for k in range(cols // 16):
      ref[r, pl.ds(k * 16, 16)] = jnp.zeros((16,), jnp.float32)
    return carry
  lax.fori_loop(0, rows, body, 0)


def _msgpass_body(x_hbm, src_hbm, dst_hbm, out_hbm,
                  sidx, didx, rows, stage, acc, sem):
  c = lax.axis_index("c")
  s = lax.axis_index("s")
  w = s * _NC + c

  _zero_vmem_2d(stage, _RPT, _D)
  pltpu.sync_copy(stage, acc.at[pl.ds(s * _RPT, _RPT)])
  plsc.subcore_barrier()

  def step(j, carry):
    e0 = (j * _NW + w) * _CHUNK
    pltpu.sync_copy(src_hbm.at[pl.ds(e0, _CHUNK)], sidx)
    pltpu.async_copy(x_hbm.at[sidx], rows, sem).wait()
    pltpu.sync_copy(dst_hbm.at[pl.ds(e0, _CHUNK)], didx)
    pltpu.sync_copy(rows, acc.at[didx], add=True)
    return carry

  lax.fori_loop(0, _FULL_ITERS, step, 0)

  @pl.when(w < _REM)
  def _():
    step(_FULL_ITERS, 0)

  plsc.subcore_barrier()
  pltpu.sync_copy(acc.at[pl.ds(s * _RPT, _RPT)], stage)
  pltpu.sync_copy(stage, out_hbm.at[c, pl.ds(s * _RPT, _RPT)])


@functools.lru_cache(maxsize=None)
def _msgpass_call():
  mesh = plsc.VectorSubcoreMesh(core_axis_name="c", subcore_axis_name="s")
  return pl.kernel(
      _msgpass_body,
      out_type=jax.ShapeDtypeStruct((_NC, _N, _D), jnp.float32),
      mesh=mesh,
      scratch_types=[
          pltpu.VMEM((_CHUNK,), jnp.int32),
          pltpu.VMEM((_CHUNK,), jnp.int32),
          pltpu.VMEM((_CHUNK, _D), jnp.float32),
          pltpu.VMEM((_RPT, _D), jnp.float32),
          pltpu.VMEM_SHARED((_N, _D), jnp.float32),
          pltpu.SemaphoreType.DMA,
      ],
  )


def _deg_body(src_hbm, dst_hbm, oin_hbm, oout_hbm,
              sidx, didx, ones, stage, acc_in, acc_out):
  c = lax.axis_index("c")
  s = lax.axis_index("s")
  w = s * _NC + c

  lane0 = jnp.where(
      lax.broadcasted_iota(jnp.int32, (16,), 0) == 0, 1.0, 0.0
  ).astype(jnp.float32)
  def fill(r, carry):
    ones[r, pl.ds(0, 16)] = lane0
    return carry
  lax.fori_loop(0, _CHUNK, fill, 0)

  _zero_vmem_2d(stage, _RPT, 16)
  pltpu.sync_copy(stage, acc_in.at[pl.ds(s * _RPT, _RPT)])
  pltpu.sync_copy(stage, acc_out.at[pl.ds(s * _RPT, _RPT)])
  plsc.subcore_barrier()

  def step(j, carry):
    e0 = (j * _NW + w) * _CHUNK
    pltpu.sync_copy(src_hbm.at[pl.ds(e0, _CHUNK)], sidx)
    pltpu.sync_copy(dst_hbm.at[pl.ds(e0, _CHUNK)], didx)
    pltpu.sync_copy(ones, acc_out.at[sidx], add=True)
    pltpu.sync_copy(ones, acc_in.at[didx], add=True)
    return carry

  lax.fori_loop(0, _FULL_ITERS, step, 0)

  @pl.when(w < _REM)
  def _():
    step(_FULL_ITERS, 0)

  plsc.subcore_barrier()
  pltpu.sync_copy(acc_in.at[pl.ds(s * _RPT, _RPT)], stage)
  pltpu.sync_copy(stage, oin_hbm.at[c, pl.ds(s * _RPT, _RPT)])
  pltpu.sync_copy(acc_out.at[pl.ds(s * _RPT, _RPT)], stage)
  pltpu.sync_copy(stage, oout_hbm.at[c, pl.ds(s * _RPT, _RPT)])


@functools.lru_cache(maxsize=None)
def _deg_call():
  mesh = plsc.VectorSubcoreMesh(core_axis_name="c", subcore_axis_name="s")
  return pl.kernel(
      _deg_body,
      out_type=(
          jax.ShapeDtypeStruct((_NC, _N, 16), jnp.float32),
          jax.ShapeDtypeStruct((_NC, _N, 16), jnp.float32),
      ),
      mesh=mesh,
      scratch_types=[
          pltpu.VMEM((_CHUNK,), jnp.int32),
          pltpu.VMEM((_CHUNK,), jnp.int32),
          pltpu.VMEM((_CHUNK, 16), jnp.float32),
          pltpu.VMEM((_RPT, 16), jnp.float32),
          pltpu.VMEM_SHARED((_N, 16), jnp.float32),
          pltpu.VMEM_SHARED((_N, 16), jnp.float32),
      ],
  )


def _prep_body(nodes, wemb, bemb, din_p, dout_p,
               h0_ref, x0_ref, inrs_ref, outrs_ref):
  h0 = jnp.dot(nodes[...], wemb[...],
               preferred_element_type=jnp.float32) + bemb[...]
  din = din_p[0, :, 0:1] + din_p[1, :, 0:1]
  dout = dout_p[0, :, 0:1] + dout_p[1, :, 0:1]
  inrs = lax.rsqrt(jnp.maximum(din, 1.0))
  outrs = lax.rsqrt(jnp.maximum(dout, 1.0))
  h0_ref[...] = h0
  x0_ref[...] = h0 * outrs
  inrs_ref[...] = inrs
  outrs_ref[...] = outrs


@functools.lru_cache(maxsize=None)
def _prep_call():
  return pl.pallas_call(
      _prep_body,
      out_shape=(
          jax.ShapeDtypeStruct((_N, _D), jnp.float32),
          jax.ShapeDtypeStruct((_N, _D), jnp.float32),
          jax.ShapeDtypeStruct((_N, 1), jnp.float32),
          jax.ShapeDtypeStruct((_N, 1), jnp.float32),
      ),
  )


def _layer_body(p, inrs, hin, w, b, snorm, gamma, beta, outrs,
                hout_ref, xnext_ref):
  agg = (p[0] + p[1]) * inrs[...]
  x = jnp.dot(agg, w[...], preferred_element_type=jnp.float32) + b[...]
  x = x * snorm[...]
  mu = jnp.mean(x, axis=0, keepdims=True)
  d = x - mu
  var = jnp.mean(d * d, axis=0, keepdims=True)
  x = gamma[...] * d * lax.rsqrt(var + 1e-5) + beta[...]
  x = jnp.maximum(x, 0.0)
  h = hin[...] + x
  hout_ref[...] = h
  xnext_ref[...] = h * outrs[...]


@functools.lru_cache(maxsize=None)
def _layer_call():
  return pl.pallas_call(
      _layer_body,
      out_shape=(
          jax.ShapeDtypeStruct((_N, _D), jnp.float32),
          jax.ShapeDtypeStruct((_N, _D), jnp.float32),
      ),
  )


def _readout_body(h, gid, w1, b1, w2, b2, out_ref):
  ids = gid[...]
  gi = lax.broadcasted_iota(jnp.int32, (_N, _G), 1)
  m = (gi == ids).astype(jnp.float32)
  counts = jnp.sum(m, axis=0)
  sums = lax.dot_general(m, h[...], (((0,), (0,)), ((), ())),
                         preferred_element_type=jnp.float32)
  hg = sums / jnp.maximum(counts, 1.0)[:, None]
  y = jnp.maximum(
      jnp.dot(hg, w1[...], preferred_element_type=jnp.float32) + b1[...], 0.0)
  out_ref[...] = jnp.dot(y, w2[...],
                         preferred_element_type=jnp.float32) + b2[...]


@functools.lru_cache(maxsize=None)
def _readout_call():
  return pl.pallas_call(
      _readout_body,
      out_shape=jax.ShapeDtypeStruct((_G, _D), jnp.float32),
  )


def kernel(nodes_feat, edge_index, edges_feat, nodes_num_norm_sqrt,
           edges_num_norm_sqrt, node_graph_ids, W_emb, b_emb, gcn_W, gcn_b,
           bn_gamma, bn_beta, W1, b1, W2, b2):
  del edges_feat, edges_num_norm_sqrt
  src = edge_index[0]
  dst = edge_index[1]

  din_p, dout_p = _deg_call()(src, dst)
  h, x, inrs, outrs = _prep_call()(
      nodes_feat, W_emb, b_emb.reshape(1, _D), din_p, dout_p)

  for i in range(_NLAYERS):
    p = _msgpass_call()(x, src, dst)
    h, x = _layer_call()(
        p, inrs, h, gcn_W[i], gcn_b[i].reshape(1, _D), nodes_num_norm_sqrt,
        bn_gamma[i].reshape(1, _D), bn_beta[i].reshape(1, _D), outrs)

  w2p = jnp.pad(W2, ((0, 0), (0, _D - W2.shape[1])))
  b2p = jnp.pad(b2, (0, _D - b2.shape[0])).reshape(1, _D)
  logits = _readout_call()(
      h, node_graph_ids.reshape(_N, 1), W1, b1.reshape(1, -1), w2p, b2p)
  return logits[:, :b2.shape[0]]


# speedup vs baseline: 3.5583x; 3.5583x over previous
import functools

import jax
import jax.numpy as jnp
from jax import lax
from jax.experimental import pallas as pl
from jax.experimental.pallas import tpu as pltpu
import jax.experimental.pallas.tpu_sc as plsc

_N = 10000
_E = 320000
_D = 128
_G = 128
_NLAYERS = 3

_NC = 2
_NS = 16
_NW = _NC * _NS
_CHUNK = 128
_NCHUNK = _E // _CHUNK
_NPAD = 10240
_RPT = _NPAD // _NS
_DSUB = 64


def _zero_vmem_2d(ref, rows, cols):
  def body(r, carry):
    for k in range(cols // 16):
      ref[r, pl.ds(k * 16, 16)] = jnp.zeros((16,), jnp.float32)
    return carry
  lax.fori_loop(0, rows, body, 0)


def _n_iters(w, nworkers):
  full = _NCHUNK // nworkers
  rem = _NCHUNK - full * nworkers
  return full + jnp.where(w < rem, 1, 0)


def _msgpass_body(x_hbm, src_hbm, dst_hbm, out_hbm,
                  sidx, didx, rows, stage, acc, sem):
  c = lax.axis_index("c")
  s = lax.axis_index("s")
  w = s * _NC + c

  _zero_vmem_2d(stage, _RPT, _DSUB)
  pltpu.sync_copy(stage, acc.at[pl.ds(s * _RPT, _RPT)])
  plsc.subcore_barrier()

  def step(j, carry):
    e0 = (j * _NW + w) * _CHUNK
    pltpu.sync_copy(src_hbm.at[pl.ds(e0, _CHUNK)], sidx)
    pltpu.async_copy(x_hbm.at[sidx], rows, sem).wait()
    pltpu.sync_copy(dst_hbm.at[pl.ds(e0, _CHUNK)], didx)
    pltpu.sync_copy(rows, acc.at[didx], add=True)
    return carry

  lax.fori_loop(0, _n_iters(w, _NW), step, 0)
  plsc.subcore_barrier()

  pltpu.sync_copy(acc.at[pl.ds(s * _RPT, _RPT)], stage)
  pltpu.sync_copy(stage, out_hbm.at[pl.ds((c * _NS + s) * _RPT, _RPT)])


@functools.lru_cache(maxsize=None)
def _msgpass_call():
  mesh = plsc.VectorSubcoreMesh(
      core_axis_name="c", subcore_axis_name="s", num_cores=_NC)
  return pl.kernel(
      _msgpass_body,
      out_type=jax.ShapeDtypeStruct((_NC * _NPAD, _DSUB), jnp.float32),
      mesh=mesh,
      scratch_types=[
          pltpu.VMEM((_CHUNK,), jnp.int32),
          pltpu.VMEM((_CHUNK,), jnp.int32),
          pltpu.VMEM((_CHUNK, _DSUB), jnp.float32),
          pltpu.VMEM((_RPT, _DSUB), jnp.float32),
          pltpu.VMEM_SHARED((_NPAD, _DSUB), jnp.float32),
          pltpu.SemaphoreType.DMA,
      ],
      compiler_params=pltpu.CompilerParams(use_tc_tiling_on_sc=False),
  )


def _deg_body(src_hbm, dst_hbm, oin_hbm, oout_hbm,
              sidx, didx, hin, hout):
  s = lax.axis_index("s")

  def zbody(r, carry):
    hin[pl.ds(r * 16, 16)] = jnp.zeros((16,), jnp.float32)
    hout[pl.ds(r * 16, 16)] = jnp.zeros((16,), jnp.float32)
    return carry
  lax.fori_loop(0, _NPAD // 16, zbody, 0)

  ones16 = jnp.ones((16,), jnp.float32)

  def step(j, carry):
    e0 = (j * _NS + s) * _CHUNK
    pltpu.sync_copy(src_hbm.at[pl.ds(e0, _CHUNK)], sidx)
    pltpu.sync_copy(dst_hbm.at[pl.ds(e0, _CHUNK)], didx)
    for k in range(_CHUNK // 16):
      plsc.addupdate_scatter(hout, [sidx[pl.ds(k * 16, 16)]], ones16)
      plsc.addupdate_scatter(hin, [didx[pl.ds(k * 16, 16)]], ones16)
    return carry

  lax.fori_loop(0, _n_iters(s, _NS), step, 0)

  pltpu.sync_copy(hin, oin_hbm.at[s])
  pltpu.sync_copy(hout, oout_hbm.at[s])


@functools.lru_cache(maxsize=None)
def _deg_call():
  mesh = plsc.VectorSubcoreMesh(
      core_axis_name="c", subcore_axis_name="s", num_cores=1)
  return pl.kernel(
      _deg_body,
      out_type=(
          jax.ShapeDtypeStruct((_NS, _NPAD), jnp.float32),
          jax.ShapeDtypeStruct((_NS, _NPAD), jnp.float32),
      ),
      mesh=mesh,
      scratch_types=[
          pltpu.VMEM((_CHUNK,), jnp.int32),
          pltpu.VMEM((_CHUNK,), jnp.int32),
          pltpu.VMEM((_NPAD,), jnp.float32),
          pltpu.VMEM((_NPAD,), jnp.float32),
      ],
      compiler_params=pltpu.CompilerParams(needs_layout_passes=False),
  )


def _prep_body(nodes, wemb, bemb, din_p, dout_p,
               h0_ref, xlo_ref, xhi_ref, inrs_ref, outrs_ref):
  h0 = jnp.dot(nodes[...], wemb[...],
               preferred_element_type=jnp.float32) + bemb[...]
  din = jnp.sum(din_p[...], axis=0)[0:_N, None]
  dout = jnp.sum(dout_p[...], axis=0)[0:_N, None]
  inrs = lax.rsqrt(jnp.maximum(din, 1.0))
  outrs = lax.rsqrt(jnp.maximum(dout, 1.0))
  x0 = h0 * outrs
  h0_ref[...] = h0
  xlo_ref[...] = x0[:, 0:_DSUB]
  xhi_ref[...] = x0[:, _DSUB:_D]
  inrs_ref[...] = inrs
  outrs_ref[...] = outrs


@functools.lru_cache(maxsize=None)
def _prep_call():
  return pl.pallas_call(
      _prep_body,
      out_shape=(
          jax.ShapeDtypeStruct((_N, _D), jnp.float32),
          jax.ShapeDtypeStruct((_N, _DSUB), jnp.float32),
          jax.ShapeDtypeStruct((_N, _DSUB), jnp.float32),
          jax.ShapeDtypeStruct((_N, 1), jnp.float32),
          jax.ShapeDtypeStruct((_N, 1), jnp.float32),
      ),
      compiler_params=pltpu.CompilerParams(
          vmem_limit_bytes=100 * 1024 * 1024),
  )


def _layer_body(plo, phi, inrs, hin, w, b, snorm, gamma, beta, outrs,
                hout_ref, xlo_ref, xhi_ref):
  agg = jnp.concatenate(
      [plo[0:_N] + plo[_NPAD:_NPAD + _N],
       phi[0:_N] + phi[_NPAD:_NPAD + _N]], axis=1)
  agg = agg * inrs[...]
  x = jnp.dot(agg, w[...], preferred_element_type=jnp.float32) + b[...]
  x = x * snorm[...]
  mu = jnp.mean(x, axis=0, keepdims=True)
  d = x - mu
  var = jnp.mean(d * d, axis=0, keepdims=True)
  x = gamma[...] * d * lax.rsqrt(var + 1e-5) + beta[...]
  x = jnp.maximum(x, 0.0)
  h = hin[...] + x
  hout_ref[...] = h
  xn = h * outrs[...]
  xlo_ref[...] = xn[:, 0:_DSUB]
  xhi_ref[...] = xn[:, _DSUB:_D]


@functools.lru_cache(maxsize=None)
def _layer_call():
  return pl.pallas_call(
      _layer_body,
      out_shape=(
          jax.ShapeDtypeStruct((_N, _D), jnp.float32),
          jax.ShapeDtypeStruct((_N, _DSUB), jnp.float32),
          jax.ShapeDtypeStruct((_N, _DSUB), jnp.float32),
      ),
      compiler_params=pltpu.CompilerParams(
          vmem_limit_bytes=100 * 1024 * 1024),
  )


def _readout_body(h, gid, w1, b1, w2, b2, out_ref):
  ids = gid[...]
  gi = lax.broadcasted_iota(jnp.int32, (_N, _G), 1)
  m = (gi == ids).astype(jnp.float32)
  counts = jnp.sum(m, axis=0)
  sums = lax.dot_general(m, h[...], (((0,), (0,)), ((), ())),
                         preferred_element_type=jnp.float32)
  hg = sums / jnp.maximum(counts, 1.0)[:, None]
  y = jnp.maximum(
      jnp.dot(hg, w1[...], preferred_element_type=jnp.float32) + b1[...], 0.0)
  out_ref[...] = jnp.dot(y, w2[...],
                         preferred_element_type=jnp.float32) + b2[...]


@functools.lru_cache(maxsize=None)
def _readout_call():
  return pl.pallas_call(
      _readout_body,
      out_shape=jax.ShapeDtypeStruct((_G, _D), jnp.float32),
      compiler_params=pltpu.CompilerParams(
          vmem_limit_bytes=100 * 1024 * 1024),
  )


def kernel(nodes_feat, edge_index, edges_feat, nodes_num_norm_sqrt,
           edges_num_norm_sqrt, node_graph_ids, W_emb, b_emb, gcn_W, gcn_b,
           bn_gamma, bn_beta, W1, b1, W2, b2):
  del edges_feat, edges_num_norm_sqrt
  src = edge_index[0]
  dst = edge_index[1]

  din_p, dout_p = _deg_call()(src, dst)
  h, xlo, xhi, inrs, outrs = _prep_call()(
      nodes_feat, W_emb, b_emb.reshape(1, _D), din_p, dout_p)

  for i in range(_NLAYERS):
    plo = _msgpass_call()(xlo, src, dst)
    phi = _msgpass_call()(xhi, src, dst)
    h, xlo, xhi = _layer_call()(
        plo, phi, inrs, h, gcn_W[i], gcn_b[i].reshape(1, _D),
        nodes_num_norm_sqrt, bn_gamma[i].reshape(1, _D),
        bn_beta[i].reshape(1, _D), outrs)

  w2p = jnp.pad(W2, ((0, 0), (0, _D - W2.shape[1])))
  b2p = jnp.pad(b2, (0, _D - b2.shape[0])).reshape(1, _D)
  logits = _readout_call()(
      h, node_graph_ids.reshape(_N, 1), W1, b1.reshape(1, -1), w2p, b2p)
  return logits[:, :b2.shape[0]]


# speedup vs baseline: 4.2075x; 1.1824x over previous
import functools

import jax
import jax.numpy as jnp
from jax import lax
from jax.experimental import pallas as pl
from jax.experimental.pallas import tpu as pltpu
import jax.experimental.pallas.tpu_sc as plsc

_N = 10000
_E = 320000
_D = 128
_G = 128
_NLAYERS = 3

_NC = 2
_NS = 16
_CHUNK = 128
_NCHUNK = _E // _CHUNK
_NPAD = 10240
_RPT = _NPAD // _NS
_DSUB = 64
_EPAD = 327680
_CHT = _EPAD // _CHUNK
_CPT = _CHT // _NS


def _zero_vmem_2d(ref, rows, cols):
  def body(r, carry):
    for k in range(cols // 16):
      ref[r, pl.ds(k * 16, 16)] = jnp.zeros((16,), jnp.float32)
    return carry
  lax.fori_loop(0, rows, body, 0)


def _n_iters(w, nworkers):
  full = _NCHUNK // nworkers
  rem = _NCHUNK - full * nworkers
  return full + jnp.where(w < rem, 1, 0)


def _msgpass_body(x_hbm, srcr_hbm, dstr_hbm, out_hbm,
                  sidx, didx0, didx1, rows0, rows1, stage, acc,
                  g0, g1, d0, d1):
  c = lax.axis_index("c")
  s = lax.axis_index("s")

  _zero_vmem_2d(stage, _RPT, _DSUB)
  pltpu.sync_copy(stage, acc.at[pl.ds(s * _RPT, _RPT)])

  pltpu.sync_copy(srcr_hbm.at[pl.ds((c * _CHT + s * _CPT), _CPT)], sidx)
  plsc.subcore_barrier()

  def gstart(j, buf, sem):
    pltpu.make_async_copy(x_hbm.at[sidx.at[j]], buf, sem).start()

  def gwait(j, buf, sem):
    pltpu.make_async_copy(x_hbm.at[sidx.at[j]], buf, sem).wait()

  def dstart(j, buf, sem):
    pltpu.make_async_copy(dstr_hbm.at[pl.ds(s * _CPT + j, 1)], buf, sem).start()

  def dwait(j, buf, sem):
    pltpu.make_async_copy(dstr_hbm.at[pl.ds(s * _CPT + j, 1)], buf, sem).wait()

  def scat(j, buf, dbuf):
    pltpu.sync_copy(buf, acc.at[dbuf.at[0]], add=True)

  gstart(0, rows0, g0)
  dstart(0, didx0, d0)

  def body(p, carry):
    j0 = p * 2
    gstart(j0 + 1, rows1, g1)
    dstart(j0 + 1, didx1, d1)
    gwait(j0, rows0, g0)
    dwait(j0, didx0, d0)
    scat(j0, rows0, didx0)
    gstart(j0 + 2, rows0, g0)
    dstart(j0 + 2, didx0, d0)
    gwait(j0 + 1, rows1, g1)
    dwait(j0 + 1, didx1, d1)
    scat(j0 + 1, rows1, didx1)
    return carry

  lax.fori_loop(0, _CPT // 2 - 1, body, 0)

  gstart(_CPT - 1, rows1, g1)
  dstart(_CPT - 1, didx1, d1)
  gwait(_CPT - 2, rows0, g0)
  dwait(_CPT - 2, didx0, d0)
  scat(_CPT - 2, rows0, didx0)
  gwait(_CPT - 1, rows1, g1)
  dwait(_CPT - 1, didx1, d1)
  scat(_CPT - 1, rows1, didx1)

  plsc.subcore_barrier()
  pltpu.sync_copy(acc.at[pl.ds(s * _RPT, _RPT)], stage)
  pltpu.sync_copy(stage, out_hbm.at[pl.ds((c * _NS + s) * _RPT, _RPT)])


@functools.lru_cache(maxsize=None)
def _msgpass_call():
  mesh = plsc.VectorSubcoreMesh(
      core_axis_name="c", subcore_axis_name="s", num_cores=_NC)
  return pl.kernel(
      _msgpass_body,
      out_type=jax.ShapeDtypeStruct((_NC * _NPAD, _DSUB), jnp.float32),
      mesh=mesh,
      scratch_types=[
          pltpu.VMEM((_CPT, _CHUNK), jnp.int32),
          pltpu.VMEM((1, _CHUNK), jnp.int32),
          pltpu.VMEM((1, _CHUNK), jnp.int32),
          pltpu.VMEM((_CHUNK, _DSUB), jnp.float32),
          pltpu.VMEM((_CHUNK, _DSUB), jnp.float32),
          pltpu.VMEM((_RPT, _DSUB), jnp.float32),
          pltpu.VMEM_SHARED((_NPAD, _DSUB), jnp.float32),
          pltpu.SemaphoreType.DMA,
          pltpu.SemaphoreType.DMA,
          pltpu.SemaphoreType.DMA,
          pltpu.SemaphoreType.DMA,
      ],
      compiler_params=pltpu.CompilerParams(use_tc_tiling_on_sc=False),
  )


def _deg_body(src_hbm, dst_hbm, oin_hbm, oout_hbm,
              sidx, didx, hin, hout):
  s = lax.axis_index("s")

  def zbody(r, carry):
    hin[pl.ds(r * 16, 16)] = jnp.zeros((16,), jnp.float32)
    hout[pl.ds(r * 16, 16)] = jnp.zeros((16,), jnp.float32)
    return carry
  lax.fori_loop(0, _NPAD // 16, zbody, 0)

  ones16 = jnp.ones((16,), jnp.float32)

  def step(j, carry):
    e0 = (j * _NS + s) * _CHUNK
    pltpu.sync_copy(src_hbm.at[pl.ds(e0, _CHUNK)], sidx)
    pltpu.sync_copy(dst_hbm.at[pl.ds(e0, _CHUNK)], didx)
    for k in range(_CHUNK // 16):
      plsc.addupdate_scatter(hout, [sidx[pl.ds(k * 16, 16)]], ones16)
      plsc.addupdate_scatter(hin, [didx[pl.ds(k * 16, 16)]], ones16)
    return carry

  lax.fori_loop(0, _n_iters(s, _NS), step, 0)

  pltpu.sync_copy(hin, oin_hbm.at[s])
  pltpu.sync_copy(hout, oout_hbm.at[s])


@functools.lru_cache(maxsize=None)
def _deg_call():
  mesh = plsc.VectorSubcoreMesh(
      core_axis_name="c", subcore_axis_name="s", num_cores=1)
  return pl.kernel(
      _deg_body,
      out_type=(
          jax.ShapeDtypeStruct((_NS, _NPAD), jnp.float32),
          jax.ShapeDtypeStruct((_NS, _NPAD), jnp.float32),
      ),
      mesh=mesh,
      scratch_types=[
          pltpu.VMEM((_CHUNK,), jnp.int32),
          pltpu.VMEM((_CHUNK,), jnp.int32),
          pltpu.VMEM((_NPAD,), jnp.float32),
          pltpu.VMEM((_NPAD,), jnp.float32),
      ],
      compiler_params=pltpu.CompilerParams(needs_layout_passes=False),
  )


def _prep_body(nodes, wemb, bemb, din_p, dout_p,
               h0_ref, xcat_ref, inrs_ref, outrs_ref):
  h0 = jnp.dot(nodes[...], wemb[...],
               preferred_element_type=jnp.float32) + bemb[...]
  din = jnp.sum(din_p[...], axis=0)[0:_N, None]
  dout = jnp.sum(dout_p[...], axis=0)[0:_N, None]
  inrs = lax.rsqrt(jnp.maximum(din, 1.0))
  outrs = lax.rsqrt(jnp.maximum(dout, 1.0))
  x0 = h0 * outrs
  h0_ref[...] = h0
  xcat_ref[0:_N] = x0[:, 0:_DSUB]
  xcat_ref[_N:2 * _N] = x0[:, _DSUB:_D]
  inrs_ref[...] = inrs
  outrs_ref[...] = outrs


@functools.lru_cache(maxsize=None)
def _prep_call():
  return pl.pallas_call(
      _prep_body,
      out_shape=(
          jax.ShapeDtypeStruct((_N, _D), jnp.float32),
          jax.ShapeDtypeStruct((2 * _N, _DSUB), jnp.float32),
          jax.ShapeDtypeStruct((_N, 1), jnp.float32),
          jax.ShapeDtypeStruct((_N, 1), jnp.float32),
      ),
      compiler_params=pltpu.CompilerParams(
          vmem_limit_bytes=100 * 1024 * 1024),
  )


def _layer_body(p, inrs, hin, w, b, snorm, gamma, beta, outrs,
                hout_ref, xcat_ref):
  agg = jnp.concatenate(
      [p[0:_N], p[_NPAD:_NPAD + _N]], axis=1)
  agg = agg * inrs[...]
  x = jnp.dot(agg, w[...], preferred_element_type=jnp.float32) + b[...]
  x = x * snorm[...]
  mu = jnp.mean(x, axis=0, keepdims=True)
  d = x - mu
  var = jnp.mean(d * d, axis=0, keepdims=True)
  x = gamma[...] * d * lax.rsqrt(var + 1e-5) + beta[...]
  x = jnp.maximum(x, 0.0)
  h = hin[...] + x
  hout_ref[...] = h
  xn = h * outrs[...]
  xcat_ref[0:_N] = xn[:, 0:_DSUB]
  xcat_ref[_N:2 * _N] = xn[:, _DSUB:_D]


@functools.lru_cache(maxsize=None)
def _layer_call():
  return pl.pallas_call(
      _layer_body,
      out_shape=(
          jax.ShapeDtypeStruct((_N, _D), jnp.float32),
          jax.ShapeDtypeStruct((2 * _N, _DSUB), jnp.float32),
      ),
      compiler_params=pltpu.CompilerParams(
          vmem_limit_bytes=100 * 1024 * 1024),
  )


def _readout_body(h, gid, w1, b1, w2, b2, out_ref):
  ids = gid[...]
  gi = lax.broadcasted_iota(jnp.int32, (_N, _G), 1)
  m = (gi == ids).astype(jnp.float32)
  counts = jnp.sum(m, axis=0)
  sums = lax.dot_general(m, h[...], (((0,), (0,)), ((), ())),
                         preferred_element_type=jnp.float32)
  hg = sums / jnp.maximum(counts, 1.0)[:, None]
  y = jnp.maximum(
      jnp.dot(hg, w1[...], preferred_element_type=jnp.float32) + b1[...], 0.0)
  out_ref[...] = jnp.dot(y, w2[...],
                         preferred_element_type=jnp.float32) + b2[...]


@functools.lru_cache(maxsize=None)
def _readout_call():
  return pl.pallas_call(
      _readout_body,
      out_shape=jax.ShapeDtypeStruct((_G, _D), jnp.float32),
      compiler_params=pltpu.CompilerParams(
          vmem_limit_bytes=100 * 1024 * 1024),
  )


def kernel(nodes_feat, edge_index, edges_feat, nodes_num_norm_sqrt,
           edges_num_norm_sqrt, node_graph_ids, W_emb, b_emb, gcn_W, gcn_b,
           bn_gamma, bn_beta, W1, b1, W2, b2):
  del edges_feat, edges_num_norm_sqrt
  src = edge_index[0]
  dst = edge_index[1]

  npad_e = _EPAD - _E
  src_pad = jnp.concatenate([src, jnp.zeros((npad_e,), jnp.int32)])
  dst_pad = jnp.concatenate(
      [dst, jnp.full((npad_e,), _NPAD - 1, jnp.int32)])
  srcr = jnp.concatenate([src_pad, src_pad + _N]).reshape(2 * _CHT, _CHUNK)
  dstr = dst_pad.reshape(_CHT, _CHUNK)

  din_p, dout_p = _deg_call()(src, dst)
  h, xcat, inrs, outrs = _prep_call()(
      nodes_feat, W_emb, b_emb.reshape(1, _D), din_p, dout_p)

  for i in range(_NLAYERS):
    p = _msgpass_call()(xcat, srcr, dstr)
    h, xcat = _layer_call()(
        p, inrs, h, gcn_W[i], gcn_b[i].reshape(1, _D),
        nodes_num_norm_sqrt, bn_gamma[i].reshape(1, _D),
        bn_beta[i].reshape(1, _D), outrs)

  w2p = jnp.pad(W2, ((0, 0), (0, _D - W2.shape[1])))
  b2p = jnp.pad(b2, (0, _D - b2.shape[0])).reshape(1, _D)
  logits = _readout_call()(
      h, node_graph_ids.reshape(_N, 1), W1, b1.reshape(1, -1), w2p, b2p)
  return logits[:, :b2.shape[0]]


# speedup vs baseline: 4.2872x; 1.0190x over previous
import functools

import jax
import jax.numpy as jnp
from jax import lax
from jax.experimental import pallas as pl
from jax.experimental.pallas import tpu as pltpu
import jax.experimental.pallas.tpu_sc as plsc

_N = 10000
_E = 320000
_D = 128
_G = 128
_NLAYERS = 3

_NC = 2
_NS = 16
_CHUNK = 128
_NCHUNK = _E // 128
_NPAD = 10240
_RPT = _NPAD // _NS
_DSUB = 64
_EPAD = 327680
_CHT = _EPAD // _CHUNK
_CPT = _CHT // _NS


def _zero_vmem_2d(ref, rows, cols):
  def body(r, carry):
    for k in range(cols // 16):
      ref[r, pl.ds(k * 16, 16)] = jnp.zeros((16,), jnp.float32)
    return carry
  lax.fori_loop(0, rows, body, 0)


def _n_iters(w, nworkers):
  full = _NCHUNK // nworkers
  rem = _NCHUNK - full * nworkers
  return full + jnp.where(w < rem, 1, 0)


def _msgpass_body(x_hbm, srcr_hbm, dstr_hbm, out_hbm, *scr):
  rows = list(scr[0:8])
  didx = list(scr[8:16])
  sidx = scr[16]
  gsem = list(scr[17:25])
  dsem = list(scr[25:33])
  tsem = list(scr[33:37])

  c = lax.axis_index("c")
  s = lax.axis_index("s")

  _zero_vmem_2d(rows[0], _CHUNK, _DSUB)
  for k in range(_RPT // _CHUNK):
    pltpu.sync_copy(rows[0],
                    acc_ref(scr).at[pl.ds(s * _RPT + k * _CHUNK, _CHUNK)])

  pltpu.sync_copy(srcr_hbm.at[pl.ds((c * _CHT + s * _CPT), _CPT)], sidx)

  acc = acc_ref(scr)

  def gstart(j, q):
    pltpu.make_async_copy(x_hbm.at[sidx.at[j]], rows[q], gsem[q]).start()

  def gwait(j, q):
    pltpu.make_async_copy(x_hbm.at[sidx.at[j]], rows[q], gsem[q]).wait()

  def dstart(j, q):
    pltpu.make_async_copy(
        dstr_hbm.at[pl.ds(s * _CPT + j, 1)], didx[q], dsem[q]).start()

  def dwait(j, q):
    pltpu.make_async_copy(
        dstr_hbm.at[pl.ds(s * _CPT + j, 1)], didx[q], dsem[q]).wait()

  def sstart(q):
    pltpu.async_copy(rows[q], acc.at[didx[q].at[0]], tsem[q % 4], add=True)

  def swait(q):
    pltpu.make_async_copy(rows[q], acc.at[didx[q].at[0]], tsem[q % 4]).wait()

  for q in range(4):
    dstart(q, q)
    gstart(q, q)

  plsc.subcore_barrier()

  def iteration(j, q, first, last):
    gwait(j, q)
    dwait(j, q)
    if not first:
      swait((q + 4) % 8)
    sstart(q)
    if not last:
      gstart(j + 4, (q + 4) % 8)
      dstart(j + 4, (q + 4) % 8)

  for j in range(8):
    iteration(j, j % 8, j < 4, False)

  def body(p, carry):
    for q in range(8):
      iteration(8 * p + q, q, False, False)
    return carry

  lax.fori_loop(1, _CPT // 8 - 1, body, 0)

  for j in range(_CPT - 8, _CPT):
    iteration(j, j % 8, False, j >= _CPT - 4)
  for j in range(_CPT - 4, _CPT):
    swait(j % 8)

  plsc.subcore_barrier()
  for k in range(_RPT // _CHUNK):
    pltpu.sync_copy(acc.at[pl.ds(s * _RPT + k * _CHUNK, _CHUNK)], rows[k])
    pltpu.sync_copy(
        rows[k],
        out_hbm.at[pl.ds((c * _NS + s) * _RPT + k * _CHUNK, _CHUNK)])


def acc_ref(scr):
  return scr[37]


@functools.lru_cache(maxsize=None)
def _msgpass_call():
  mesh = plsc.VectorSubcoreMesh(
      core_axis_name="c", subcore_axis_name="s", num_cores=_NC)
  scratch = (
      [pltpu.VMEM((_CHUNK, _DSUB), jnp.float32)] * 8
      + [pltpu.VMEM((1, _CHUNK), jnp.int32)] * 8
      + [pltpu.VMEM((_CPT, _CHUNK), jnp.int32)]
      + [pltpu.SemaphoreType.DMA] * 20
      + [pltpu.VMEM_SHARED((_NPAD, _DSUB), jnp.float32)]
  )
  return pl.kernel(
      _msgpass_body,
      out_type=jax.ShapeDtypeStruct((_NC * _NPAD, _DSUB), jnp.float32),
      mesh=mesh,
      scratch_types=scratch,
      compiler_params=pltpu.CompilerParams(use_tc_tiling_on_sc=False),
  )


def _deg_body(src_hbm, dst_hbm, oin_hbm, oout_hbm,
              sidx, didx, hin, hout):
  s = lax.axis_index("s")

  def zbody(r, carry):
    hin[pl.ds(r * 16, 16)] = jnp.zeros((16,), jnp.float32)
    hout[pl.ds(r * 16, 16)] = jnp.zeros((16,), jnp.float32)
    return carry
  lax.fori_loop(0, _NPAD // 16, zbody, 0)

  ones16 = jnp.ones((16,), jnp.float32)

  def step(j, carry):
    e0 = (j * _NS + s) * 128
    pltpu.sync_copy(src_hbm.at[pl.ds(e0, 128)], sidx)
    pltpu.sync_copy(dst_hbm.at[pl.ds(e0, 128)], didx)
    for k in range(128 // 16):
      plsc.addupdate_scatter(hout, [sidx[pl.ds(k * 16, 16)]], ones16)
      plsc.addupdate_scatter(hin, [didx[pl.ds(k * 16, 16)]], ones16)
    return carry

  lax.fori_loop(0, _n_iters(s, _NS), step, 0)

  pltpu.sync_copy(hin, oin_hbm.at[s])
  pltpu.sync_copy(hout, oout_hbm.at[s])


@functools.lru_cache(maxsize=None)
def _deg_call():
  mesh = plsc.VectorSubcoreMesh(
      core_axis_name="c", subcore_axis_name="s", num_cores=1)
  return pl.kernel(
      _deg_body,
      out_type=(
          jax.ShapeDtypeStruct((_NS, _NPAD), jnp.float32),
          jax.ShapeDtypeStruct((_NS, _NPAD), jnp.float32),
      ),
      mesh=mesh,
      scratch_types=[
          pltpu.VMEM((128,), jnp.int32),
          pltpu.VMEM((128,), jnp.int32),
          pltpu.VMEM((_NPAD,), jnp.float32),
          pltpu.VMEM((_NPAD,), jnp.float32),
      ],
      compiler_params=pltpu.CompilerParams(needs_layout_passes=False),
  )


def _prep_body(nodes, wemb, bemb, din_p, dout_p,
               h0_ref, xcat_ref, inrs_ref, outrs_ref):
  h0 = jnp.dot(nodes[...], wemb[...],
               preferred_element_type=jnp.float32) + bemb[...]
  din = jnp.sum(din_p[...], axis=0)[0:_N, None]
  dout = jnp.sum(dout_p[...], axis=0)[0:_N, None]
  inrs = lax.rsqrt(jnp.maximum(din, 1.0))
  outrs = lax.rsqrt(jnp.maximum(dout, 1.0))
  x0 = h0 * outrs
  h0_ref[...] = h0
  xcat_ref[0:_N] = x0[:, 0:_DSUB]
  xcat_ref[_N:2 * _N] = x0[:, _DSUB:_D]
  inrs_ref[...] = inrs
  outrs_ref[...] = outrs


@functools.lru_cache(maxsize=None)
def _prep_call():
  return pl.pallas_call(
      _prep_body,
      out_shape=(
          jax.ShapeDtypeStruct((_N, _D), jnp.float32),
          jax.ShapeDtypeStruct((2 * _N, _DSUB), jnp.float32),
          jax.ShapeDtypeStruct((_N, 1), jnp.float32),
          jax.ShapeDtypeStruct((_N, 1), jnp.float32),
      ),
      compiler_params=pltpu.CompilerParams(
          vmem_limit_bytes=100 * 1024 * 1024),
  )


def _layer_body(p, inrs, hin, w, b, snorm, gamma, beta, outrs,
                hout_ref, xcat_ref):
  agg = jnp.concatenate(
      [p[0:_N], p[_NPAD:_NPAD + _N]], axis=1)
  agg = agg * inrs[...]
  x = jnp.dot(agg, w[...], preferred_element_type=jnp.float32) + b[...]
  x = x * snorm[...]
  mu = jnp.mean(x, axis=0, keepdims=True)
  d = x - mu
  var = jnp.mean(d * d, axis=0, keepdims=True)
  x = gamma[...] * d * lax.rsqrt(var + 1e-5) + beta[...]
  x = jnp.maximum(x, 0.0)
  h = hin[...] + x
  hout_ref[...] = h
  xn = h * outrs[...]
  xcat_ref[0:_N] = xn[:, 0:_DSUB]
  xcat_ref[_N:2 * _N] = xn[:, _DSUB:_D]


@functools.lru_cache(maxsize=None)
def _layer_call():
  return pl.pallas_call(
      _layer_body,
      out_shape=(
          jax.ShapeDtypeStruct((_N, _D), jnp.float32),
          jax.ShapeDtypeStruct((2 * _N, _DSUB), jnp.float32),
      ),
      compiler_params=pltpu.CompilerParams(
          vmem_limit_bytes=100 * 1024 * 1024),
  )


def _readout_body(h, gid, w1, b1, w2, b2, out_ref):
  ids = gid[...]
  gi = lax.broadcasted_iota(jnp.int32, (_N, _G), 1)
  m = (gi == ids).astype(jnp.float32)
  counts = jnp.sum(m, axis=0)
  sums = lax.dot_general(m, h[...], (((0,), (0,)), ((), ())),
                         preferred_element_type=jnp.float32)
  hg = sums / jnp.maximum(counts, 1.0)[:, None]
  y = jnp.maximum(
      jnp.dot(hg, w1[...], preferred_element_type=jnp.float32) + b1[...], 0.0)
  out_ref[...] = jnp.dot(y, w2[...],
                         preferred_element_type=jnp.float32) + b2[...]


@functools.lru_cache(maxsize=None)
def _readout_call():
  return pl.pallas_call(
      _readout_body,
      out_shape=jax.ShapeDtypeStruct((_G, _D), jnp.float32),
      compiler_params=pltpu.CompilerParams(
          vmem_limit_bytes=100 * 1024 * 1024),
  )


def kernel(nodes_feat, edge_index, edges_feat, nodes_num_norm_sqrt,
           edges_num_norm_sqrt, node_graph_ids, W_emb, b_emb, gcn_W, gcn_b,
           bn_gamma, bn_beta, W1, b1, W2, b2):
  del edges_feat, edges_num_norm_sqrt
  src = edge_index[0]
  dst = edge_index[1]

  npad_e = _EPAD - _E
  src_pad = jnp.concatenate([src, jnp.zeros((npad_e,), jnp.int32)])
  dst_pad = jnp.concatenate(
      [dst, jnp.full((npad_e,), _NPAD - 1, jnp.int32)])
  srcr = jnp.concatenate([src_pad, src_pad + _N]).reshape(2 * _CHT, _CHUNK)
  dstr = dst_pad.reshape(_CHT, _CHUNK)

  din_p, dout_p = _deg_call()(src, dst)
  h, xcat, inrs, outrs = _prep_call()(
      nodes_feat, W_emb, b_emb.reshape(1, _D), din_p, dout_p)

  for i in range(_NLAYERS):
    p = _msgpass_call()(xcat, srcr, dstr)
    h, xcat = _layer_call()(
        p, inrs, h, gcn_W[i], gcn_b[i].reshape(1, _D),
        nodes_num_norm_sqrt, bn_gamma[i].reshape(1, _D),
        bn_beta[i].reshape(1, _D), outrs)

  w2p = jnp.pad(W2, ((0, 0), (0, _D - W2.shape[1])))
  b2p = jnp.pad(b2, (0, _D - b2.shape[0])).reshape(1, _D)
  logits = _readout_call()(
      h, node_graph_ids.reshape(_N, 1), W1, b1.reshape(1, -1), w2p, b2p)
  return logits[:, :b2.shape[0]]


# speedup vs baseline: 5.0760x; 1.1840x over previous
import functools

import jax
import jax.numpy as jnp
from jax import lax
from jax.experimental import pallas as pl
from jax.experimental.pallas import tpu as pltpu
import jax.experimental.pallas.tpu_sc as plsc

_N = 10000
_E = 320000
_D = 128
_G = 128
_NLAYERS = 3

_NC = 2
_NS = 16
_CHUNK = 128
_NCHUNK = _E // 128
_NPAD = 10240
_RPT = _NPAD // _NS
_DSUB = 64
_EPAD = 327680
_CHT = _EPAD // _CHUNK
_CPT = _CHT // _NS


def _zero_vmem_2d(ref, rows, cols):
  def body(r, carry):
    for k in range(cols // 16):
      ref[r, pl.ds(k * 16, 16)] = jnp.zeros((16,), jnp.float32)
    return carry
  lax.fori_loop(0, rows, body, 0)


def _n_iters(w, nworkers):
  full = _NCHUNK // nworkers
  rem = _NCHUNK - full * nworkers
  return full + jnp.where(w < rem, 1, 0)


def _msgpass_body(x_hbm, srcr_hbm, dstr_hbm, out_hbm, *scr):
  rows = list(scr[0:8])
  didx = list(scr[8:16])
  sidx = scr[16]
  gsem = list(scr[17:25])
  dsem = list(scr[25:33])
  tsem = list(scr[33:37])

  c = lax.axis_index("c")
  s = lax.axis_index("s")

  _zero_vmem_2d(rows[0], _CHUNK, _DSUB)
  for k in range(_RPT // _CHUNK):
    pltpu.sync_copy(rows[0],
                    acc_ref(scr).at[pl.ds(s * _RPT + k * _CHUNK, _CHUNK)])

  pltpu.sync_copy(srcr_hbm.at[pl.ds((c * _CHT + s * _CPT), _CPT)], sidx)

  acc = acc_ref(scr)

  def gstart(j, q):
    pltpu.make_async_copy(x_hbm.at[sidx.at[j]], rows[q], gsem[q]).start()

  def gwait(j, q):
    pltpu.make_async_copy(x_hbm.at[sidx.at[j]], rows[q], gsem[q]).wait()

  def dstart(j, q):
    pltpu.make_async_copy(
        dstr_hbm.at[pl.ds(s * _CPT + j, 1)], didx[q], dsem[q]).start()

  def dwait(j, q):
    pltpu.make_async_copy(
        dstr_hbm.at[pl.ds(s * _CPT + j, 1)], didx[q], dsem[q]).wait()

  def sstart(q):
    pltpu.async_copy(rows[q], acc.at[didx[q].at[0]], tsem[q % 4], add=True)

  def swait(q):
    pltpu.make_async_copy(rows[q], acc.at[didx[q].at[0]], tsem[q % 4]).wait()

  for q in range(4):
    dstart(q, q)
    gstart(q, q)

  plsc.subcore_barrier()

  def iteration(j, q, first, last):
    gwait(j, q)
    dwait(j, q)
    if not first:
      swait((q + 4) % 8)
    sstart(q)
    if not last:
      gstart(j + 4, (q + 4) % 8)
      dstart(j + 4, (q + 4) % 8)

  for j in range(8):
    iteration(j, j % 8, j < 4, False)

  def body(p, carry):
    for q in range(8):
      iteration(8 * p + q, q, False, False)
    return carry

  lax.fori_loop(1, _CPT // 8 - 1, body, 0)

  for j in range(_CPT - 8, _CPT):
    iteration(j, j % 8, False, j >= _CPT - 4)
  for j in range(_CPT - 4, _CPT):
    swait(j % 8)

  plsc.subcore_barrier()
  for k in range(_RPT // _CHUNK):
    pltpu.sync_copy(acc.at[pl.ds(s * _RPT + k * _CHUNK, _CHUNK)], rows[k])
    pltpu.sync_copy(
        rows[k],
        out_hbm.at[pl.ds((c * _NS + s) * _RPT + k * _CHUNK, _CHUNK)])


def acc_ref(scr):
  return scr[37]


@functools.lru_cache(maxsize=None)
def _msgpass_call():
  mesh = plsc.VectorSubcoreMesh(
      core_axis_name="c", subcore_axis_name="s", num_cores=_NC)
  scratch = (
      [pltpu.VMEM((_CHUNK, _DSUB), jnp.float32)] * 8
      + [pltpu.VMEM((1, _CHUNK), jnp.int32)] * 8
      + [pltpu.VMEM((_CPT, _CHUNK), jnp.int32)]
      + [pltpu.SemaphoreType.DMA] * 20
      + [pltpu.VMEM_SHARED((_NPAD, _DSUB), jnp.float32)]
  )
  return pl.kernel(
      _msgpass_body,
      out_type=jax.ShapeDtypeStruct((_NC * _NPAD, _DSUB), jnp.float32),
      mesh=mesh,
      scratch_types=scratch,
      compiler_params=pltpu.CompilerParams(use_tc_tiling_on_sc=False),
  )


def _deg_body(srcd_hbm, dstr_hbm, oin_hbm, oout_hbm,
              sidx, didx, hin, hout):
  c = lax.axis_index("c")
  s = lax.axis_index("s")
  w = c * _NS + s

  def zbody(r, carry):
    hin[pl.ds(r * 16, 16)] = jnp.zeros((16,), jnp.float32)
    hout[pl.ds(r * 16, 16)] = jnp.zeros((16,), jnp.float32)
    return carry
  lax.fori_loop(0, _NPAD // 16, zbody, 0)

  cpw = _CHT // 32
  pltpu.sync_copy(srcd_hbm.at[pl.ds(w * cpw, cpw)], sidx)
  pltpu.sync_copy(dstr_hbm.at[pl.ds(w * cpw, cpw)], didx)

  ones16 = jnp.ones((16,), jnp.float32)

  def step(j, carry):
    for k in range(_CHUNK // 16):
      plsc.addupdate_scatter(hout, [sidx[j, pl.ds(k * 16, 16)]], ones16)
      plsc.addupdate_scatter(hin, [didx[j, pl.ds(k * 16, 16)]], ones16)
    return carry

  lax.fori_loop(0, cpw, step, 0)

  pltpu.sync_copy(hin, oin_hbm.at[w])
  pltpu.sync_copy(hout, oout_hbm.at[w])


@functools.lru_cache(maxsize=None)
def _deg_call():
  mesh = plsc.VectorSubcoreMesh(
      core_axis_name="c", subcore_axis_name="s", num_cores=_NC)
  return pl.kernel(
      _deg_body,
      out_type=(
          jax.ShapeDtypeStruct((2 * _NS, _NPAD), jnp.float32),
          jax.ShapeDtypeStruct((2 * _NS, _NPAD), jnp.float32),
      ),
      mesh=mesh,
      scratch_types=[
          pltpu.VMEM((_CHT // 32, _CHUNK), jnp.int32),
          pltpu.VMEM((_CHT // 32, _CHUNK), jnp.int32),
          pltpu.VMEM((_NPAD,), jnp.float32),
          pltpu.VMEM((_NPAD,), jnp.float32),
      ],
      compiler_params=pltpu.CompilerParams(needs_layout_passes=False),
  )


def _prep_body(nodes, wemb, bemb, din_p, dout_p,
               h0_ref, xcat_ref, inrs_ref, outrs_ref):
  h0 = jnp.dot(nodes[...], wemb[...],
               preferred_element_type=jnp.float32) + bemb[...]
  din = jnp.sum(din_p[...], axis=0)[0:_N, None]
  dout = jnp.sum(dout_p[...], axis=0)[0:_N, None]
  inrs = lax.rsqrt(jnp.maximum(din, 1.0))
  outrs = lax.rsqrt(jnp.maximum(dout, 1.0))
  x0 = h0 * outrs
  h0_ref[...] = h0
  xcat_ref[0:_N] = x0[:, 0:_DSUB]
  xcat_ref[_N:2 * _N] = x0[:, _DSUB:_D]
  inrs_ref[...] = inrs
  outrs_ref[...] = outrs


@functools.lru_cache(maxsize=None)
def _prep_call():
  return pl.pallas_call(
      _prep_body,
      out_shape=(
          jax.ShapeDtypeStruct((_N, _D), jnp.float32),
          jax.ShapeDtypeStruct((2 * _N, _DSUB), jnp.float32),
          jax.ShapeDtypeStruct((_N, 1), jnp.float32),
          jax.ShapeDtypeStruct((_N, 1), jnp.float32),
      ),
      compiler_params=pltpu.CompilerParams(
          vmem_limit_bytes=100 * 1024 * 1024),
  )


def _layer_body(p, inrs, hin, w, b, snorm, gamma, beta, outrs,
                hout_ref, xcat_ref):
  agg = jnp.concatenate(
      [p[0:_N], p[_NPAD:_NPAD + _N]], axis=1)
  agg = agg * inrs[...]
  x = jnp.dot(agg, w[...], preferred_element_type=jnp.float32) + b[...]
  x = x * snorm[...]
  mu = jnp.mean(x, axis=0, keepdims=True)
  d = x - mu
  var = jnp.mean(d * d, axis=0, keepdims=True)
  x = gamma[...] * d * lax.rsqrt(var + 1e-5) + beta[...]
  x = jnp.maximum(x, 0.0)
  h = hin[...] + x
  hout_ref[...] = h
  xn = h * outrs[...]
  xcat_ref[0:_N] = xn[:, 0:_DSUB]
  xcat_ref[_N:2 * _N] = xn[:, _DSUB:_D]


@functools.lru_cache(maxsize=None)
def _layer_call():
  return pl.pallas_call(
      _layer_body,
      out_shape=(
          jax.ShapeDtypeStruct((_N, _D), jnp.float32),
          jax.ShapeDtypeStruct((2 * _N, _DSUB), jnp.float32),
      ),
      compiler_params=pltpu.CompilerParams(
          vmem_limit_bytes=100 * 1024 * 1024),
  )


def _readout_body(h, gid, w1, b1, w2, b2, out_ref):
  ids = gid[...]
  gi = lax.broadcasted_iota(jnp.int32, (_N, _G), 1)
  m = (gi == ids).astype(jnp.float32)
  counts = jnp.sum(m, axis=0)
  sums = lax.dot_general(m, h[...], (((0,), (0,)), ((), ())),
                         preferred_element_type=jnp.float32)
  hg = sums / jnp.maximum(counts, 1.0)[:, None]
  y = jnp.maximum(
      jnp.dot(hg, w1[...], preferred_element_type=jnp.float32) + b1[...], 0.0)
  out_ref[...] = jnp.dot(y, w2[...],
                         preferred_element_type=jnp.float32) + b2[...]


@functools.lru_cache(maxsize=None)
def _readout_call():
  return pl.pallas_call(
      _readout_body,
      out_shape=jax.ShapeDtypeStruct((_G, _D), jnp.float32),
      compiler_params=pltpu.CompilerParams(
          vmem_limit_bytes=100 * 1024 * 1024),
  )


def kernel(nodes_feat, edge_index, edges_feat, nodes_num_norm_sqrt,
           edges_num_norm_sqrt, node_graph_ids, W_emb, b_emb, gcn_W, gcn_b,
           bn_gamma, bn_beta, W1, b1, W2, b2):
  del edges_feat, edges_num_norm_sqrt
  src = edge_index[0]
  dst = edge_index[1]

  npad_e = _EPAD - _E
  src_pad = jnp.concatenate([src, jnp.zeros((npad_e,), jnp.int32)])
  dst_pad = jnp.concatenate(
      [dst, jnp.full((npad_e,), _NPAD - 1, jnp.int32)])
  srcr = jnp.concatenate([src_pad, src_pad + _N]).reshape(2 * _CHT, _CHUNK)
  dstr = dst_pad.reshape(_CHT, _CHUNK)

  srcd = jnp.concatenate(
      [src, jnp.full((npad_e,), _NPAD - 1, jnp.int32)]).reshape(_CHT, _CHUNK)
  din_p, dout_p = _deg_call()(srcd, dstr)
  h, xcat, inrs, outrs = _prep_call()(
      nodes_feat, W_emb, b_emb.reshape(1, _D), din_p, dout_p)

  for i in range(_NLAYERS):
    p = _msgpass_call()(xcat, srcr, dstr)
    h, xcat = _layer_call()(
        p, inrs, h, gcn_W[i], gcn_b[i].reshape(1, _D),
        nodes_num_norm_sqrt, bn_gamma[i].reshape(1, _D),
        bn_beta[i].reshape(1, _D), outrs)

  w2p = jnp.pad(W2, ((0, 0), (0, _D - W2.shape[1])))
  b2p = jnp.pad(b2, (0, _D - b2.shape[0])).reshape(1, _D)
  logits = _readout_call()(
      h, node_graph_ids.reshape(_N, 1), W1, b1.reshape(1, -1), w2p, b2p)
  return logits[:, :b2.shape[0]]
